# baseline (speedup 1.0000x reference)
import functools

import jax
import jax.numpy as jnp
from jax import lax
from jax.experimental import pallas as pl
from jax.experimental.pallas import tpu as pltpu
from jax.experimental.pallas import tpu_sc as plsc

N_NODES = 10000
N_EDGES = 160000
D_FEAT = 256
DH = 128
NC = 2
NS = 16
L = 16
PAD_ROWS = 8
E_PAD = 163840
E_PER_TILE = E_PAD // NS
BATCH = 64
NB = E_PER_TILE // BATCH
NBUF = 4
ROWS_PER_TILE = N_NODES // NS
ZROWS = 25


def _mp_body(src4, dst4, x2, out, srcb, dstb, rows, zbuf, acc, *sems):
    gsems, isems = sems[:NBUF], sems[NBUF:]
    cid = lax.axis_index("c")
    sid = lax.axis_index("s")

    def i_start(j, q):
        pltpu.make_async_copy(src4.at[sid, j], srcb.at[q], isems[q]).start()
        pltpu.make_async_copy(dst4.at[sid, j], dstb.at[q], isems[q]).start()

    def i_wait(q):
        pltpu.make_async_copy(src4.at[sid, 0], srcb.at[q], isems[q]).wait()
        pltpu.make_async_copy(dst4.at[sid, 0], dstb.at[q], isems[q]).wait()

    def tfm(q):
        for c in range(BATCH // L):
            v = srcb[q, pl.ds(c * L, L)]
            srcb[q, pl.ds(c * L, L)] = v * 2 + cid

    def g_start(j, q):
        pltpu.make_async_copy(x2.at[srcb.at[q]], rows.at[q], gsems[q]).start()

    def g_wait(q):
        pltpu.make_async_copy(x2.at[srcb.at[0]], rows.at[q], gsems[q]).wait()

    i_start(0, 0)
    i_start(1, 1)
    i_start(2, 2)
    i_wait(0)
    tfm(0)
    g_start(0, 0)
    i_wait(1)
    tfm(1)
    g_start(1, 1)
    i_wait(2)
    tfm(2)
    g_start(2, 2)
    i_start(3, 3)

    zero = jnp.zeros((L,), jnp.float32)

    def zfill(i, _):
        for c in range(DH // L):
            zbuf[i, pl.ds(c * L, L)] = zero
        return 0

    lax.fori_loop(0, ZROWS, zfill, 0)

    def zcopy(k, _):
        pltpu.sync_copy(zbuf, acc.at[pl.ds(sid * ROWS_PER_TILE + k * ZROWS, ZROWS)])
        return 0

    lax.fori_loop(0, ROWS_PER_TILE // ZROWS, zcopy, 0)
    plsc.subcore_barrier()

    def body(jo, _):
        for u in range(NBUF):
            j = jo * NBUF + u
            g_wait(u)

            @pl.when(j + 3 < NB)
            def _():
                q = (u + 3) % NBUF
                i_wait(q)
                tfm(q)
                g_start(j + 3, q)

            pltpu.sync_copy(rows.at[u], acc.at[dstb.at[u]], add=True)

            @pl.when(j + 4 < NB)
            def _():
                i_start(j + 4, u)
        return 0

    lax.fori_loop(0, NB // NBUF, body, 0)
    plsc.subcore_barrier()

    r0 = sid * ROWS_PER_TILE
    pltpu.sync_copy(acc.at[pl.ds(r0, ROWS_PER_TILE)],
                    out.at[pl.ds(r0, ROWS_PER_TILE), cid])


_mp_kernel = functools.partial(
    pl.kernel,
    mesh=plsc.VectorSubcoreMesh(core_axis_name="c", subcore_axis_name="s"),
    out_type=jax.ShapeDtypeStruct((N_NODES, NC, DH), jnp.float32),
    scratch_types=[
        pltpu.VMEM((NBUF, BATCH), jnp.int32),
        pltpu.VMEM((NBUF, BATCH), jnp.int32),
        pltpu.VMEM((NBUF, BATCH, DH), jnp.float32),
        pltpu.VMEM((ZROWS, DH), jnp.float32),
        pltpu.VMEM_SHARED((N_NODES + PAD_ROWS, DH), jnp.float32),
    ] + [pltpu.SemaphoreType.DMA] * (2 * NBUF),
)(_mp_body)


@jax.jit
def kernel(x, edge_index):
    x2 = x.reshape(N_NODES * 2, DH)
    pad = jnp.arange(E_PAD - N_EDGES, dtype=jnp.int32)
    src4 = jnp.concatenate([edge_index[1], pad % N_NODES]).reshape(NS, NB, BATCH)
    dst4 = jnp.concatenate(
        [edge_index[0], N_NODES + pad % PAD_ROWS]).reshape(NS, NB, BATCH)
    out = _mp_kernel(src4, dst4, x2)
    return out.reshape(N_NODES, D_FEAT)


# speedup vs baseline: 1.3135x; 1.3135x over previous
import functools

import jax
import jax.numpy as jnp
from jax import lax
from jax.experimental import pallas as pl
from jax.experimental.pallas import tpu as pltpu
from jax.experimental.pallas import tpu_sc as plsc

N_NODES = 10000
N_EDGES = 160000
D_FEAT = 256
DH = 128
NC = 2
NS = 16
L = 16
PAD_ROWS = 8
E_PAD = 163840
E_PER_TILE = E_PAD // NS
BATCH = 64
NB = E_PER_TILE // BATCH
NBUF = 4
ROWS_PER_TILE = N_NODES // NS
ZROWS = 25


def _mp_body(src4, dst4, x2, out, srcb, dstb, rows, zbuf, acc, *sems):
    gsems = sems[:NBUF]
    isems = sems[NBUF:2 * NBUF]
    dsems = sems[2 * NBUF:3 * NBUF]
    ssems = sems[3 * NBUF:]
    cid = lax.axis_index("c")
    sid = lax.axis_index("s")

    def i_start(j, q):
        pltpu.make_async_copy(src4.at[sid, j], srcb.at[q], isems[q]).start()

    def i_wait(q):
        pltpu.make_async_copy(src4.at[sid, 0], srcb.at[q], isems[q]).wait()

    def d_start(j, q):
        pltpu.make_async_copy(dst4.at[sid, j], dstb.at[q], dsems[q]).start()

    def d_wait(q):
        pltpu.make_async_copy(dst4.at[sid, 0], dstb.at[q], dsems[q]).wait()

    def tfm(q):
        for c in range(BATCH // L):
            v = srcb[q, pl.ds(c * L, L)]
            srcb[q, pl.ds(c * L, L)] = v * 2 + cid

    def g_start(j, q):
        pltpu.make_async_copy(x2.at[srcb.at[q]], rows.at[q], gsems[q]).start()

    def g_wait(q):
        pltpu.make_async_copy(x2.at[srcb.at[0]], rows.at[q], gsems[q]).wait()

    def s_start(q):
        pltpu.make_async_copy(rows.at[q], acc.at[dstb.at[q]],
                              ssems[q]).start(add=True)

    def s_wait(q):
        pltpu.make_async_copy(rows.at[q], acc.at[dstb.at[q]], ssems[q]).wait()

    i_start(0, 0)
    i_start(1, 1)
    i_start(2, 2)
    d_start(0, 0)
    d_start(1, 1)
    i_wait(0)
    tfm(0)
    g_start(0, 0)
    i_wait(1)
    tfm(1)
    g_start(1, 1)

    zero = jnp.zeros((L,), jnp.float32)

    def zfill(i, _):
        for c in range(DH // L):
            zbuf[i, pl.ds(c * L, L)] = zero
        return 0

    lax.fori_loop(0, ZROWS, zfill, 0)

    def zcopy(k, _):
        pltpu.sync_copy(zbuf, acc.at[pl.ds(sid * ROWS_PER_TILE + k * ZROWS, ZROWS)])
        return 0

    lax.fori_loop(0, ROWS_PER_TILE // ZROWS, zcopy, 0)
    plsc.subcore_barrier()

    def body(jo, _):
        for u in range(NBUF):
            j = jo * NBUF + u
            g_wait(u)

            @pl.when(j + 2 < NB)
            def _():
                q = (u + 2) % NBUF
                i_wait(q)
                tfm(q)

                @pl.when(j >= 2)
                def _():
                    s_wait(q)

                g_start(j + 2, q)
                d_start(j + 2, q)

            d_wait(u)
            s_start(u)

            @pl.when(j + 3 < NB)
            def _():
                i_start(j + 3, (u + 3) % NBUF)
        return 0

    lax.fori_loop(0, NB // NBUF, body, 0)
    for u in range(NBUF):
        s_wait(u)
    plsc.subcore_barrier()

    r0 = sid * ROWS_PER_TILE
    pltpu.sync_copy(acc.at[pl.ds(r0, ROWS_PER_TILE)],
                    out.at[pl.ds(r0, ROWS_PER_TILE), cid])


_mp_kernel = functools.partial(
    pl.kernel,
    mesh=plsc.VectorSubcoreMesh(core_axis_name="c", subcore_axis_name="s"),
    out_type=jax.ShapeDtypeStruct((N_NODES, NC, DH), jnp.float32),
    scratch_types=[
        pltpu.VMEM((NBUF, BATCH), jnp.int32),
        pltpu.VMEM((NBUF, BATCH), jnp.int32),
        pltpu.VMEM((NBUF, BATCH, DH), jnp.float32),
        pltpu.VMEM((ZROWS, DH), jnp.float32),
        pltpu.VMEM_SHARED((N_NODES + PAD_ROWS, DH), jnp.float32),
    ] + [pltpu.SemaphoreType.DMA] * (4 * NBUF),
)(_mp_body)


@jax.jit
def kernel(x, edge_index):
    x2 = x.reshape(N_NODES * 2, DH)
    pad = jnp.arange(E_PAD - N_EDGES, dtype=jnp.int32)
    src4 = jnp.concatenate([edge_index[1], pad % N_NODES]).reshape(NS, NB, BATCH)
    dst4 = jnp.concatenate(
        [edge_index[0], N_NODES + pad % PAD_ROWS]).reshape(NS, NB, BATCH)
    out = _mp_kernel(src4, dst4, x2)
    return out.reshape(N_NODES, D_FEAT)


# speedup vs baseline: 1.4408x; 1.0969x over previous
import functools

import jax
import jax.numpy as jnp
from jax import lax
from jax.experimental import pallas as pl
from jax.experimental.pallas import tpu as pltpu
from jax.experimental.pallas import tpu_sc as plsc

N_NODES = 10000
N_EDGES = 160000
D_FEAT = 256
DH = 128
NC = 2
NS = 16
L = 16
PAD_ROWS = 8
E_PAD = 163840
E_PER_TILE = E_PAD // NS
BATCH = 64
NB = E_PER_TILE // BATCH
NBUF = 4
ROWS_PER_TILE = N_NODES // NS
ZROWS = 25


def _mp_body(src4, dst4, x2, out, srcb, dstb, rows, zbuf, acc, *sems):
    gsems = sems[:NBUF]
    isems = sems[NBUF:2 * NBUF]
    dsems = sems[2 * NBUF:3 * NBUF]
    ssems = sems[3 * NBUF:]
    cid = lax.axis_index("c")
    sid = lax.axis_index("s")

    def i_start(j, q):
        pltpu.make_async_copy(src4.at[sid, j], srcb.at[q], isems[q]).start()

    def i_wait(q):
        pltpu.make_async_copy(src4.at[sid, 0], srcb.at[q], isems[q]).wait()

    def d_start(j, q):
        pltpu.make_async_copy(dst4.at[sid, j], dstb.at[q], dsems[q]).start()

    def d_wait(q):
        pltpu.make_async_copy(dst4.at[sid, 0], dstb.at[q], dsems[q]).wait()

    def tfm(q):
        for c in range(BATCH // L):
            v = srcb[q, pl.ds(c * L, L)]
            srcb[q, pl.ds(c * L, L)] = v * 2 + cid

    def g_start(j, q):
        pltpu.make_async_copy(x2.at[srcb.at[q]], rows.at[q], gsems[q]).start()

    def g_wait(q):
        pltpu.make_async_copy(x2.at[srcb.at[0]], rows.at[q], gsems[q]).wait()

    def s_start(q):
        pltpu.make_async_copy(rows.at[q], acc.at[dstb.at[q]],
                              ssems[q]).start(add=True)

    def s_wait(q):
        pltpu.make_async_copy(rows.at[q], acc.at[dstb.at[q]], ssems[q]).wait()

    i_start(0, 0)
    i_start(1, 1)
    i_start(2, 2)
    d_start(0, 0)
    d_start(1, 1)
    i_wait(0)
    tfm(0)
    g_start(0, 0)
    i_wait(1)
    tfm(1)
    g_start(1, 1)

    zero = jnp.zeros((L,), jnp.float32)

    def zfill(i, _):
        for c in range(DH // L):
            zbuf[i, pl.ds(c * L, L)] = zero
        return 0

    lax.fori_loop(0, ZROWS, zfill, 0)

    def zcopy(k, _):
        pltpu.sync_copy(zbuf, acc.at[pl.ds(sid * ROWS_PER_TILE + k * ZROWS, ZROWS)])
        return 0

    lax.fori_loop(0, ROWS_PER_TILE // ZROWS, zcopy, 0)
    plsc.subcore_barrier()

    def body(jo, _):
        for u in range(NBUF):
            j = jo * NBUF + u
            g_wait(u)

            @pl.when(j + 2 < NB)
            def _():
                q = (u + 2) % NBUF
                i_wait(q)
                tfm(q)

                @pl.when(j >= 2)
                def _():
                    s_wait(q)

                g_start(j + 2, q)
                d_start(j + 2, q)

            d_wait(u)
            s_start(u)

            @pl.when(j + 3 < NB)
            def _():
                i_start(j + 3, (u + 3) % NBUF)
        return 0

    lax.fori_loop(0, NB // NBUF, body, 0)
    for u in range(NBUF):
        s_wait(u)
    plsc.subcore_barrier()

    r0 = pl.multiple_of(sid * 624, 8)
    c0 = pl.multiple_of(cid * DH, DH)
    pltpu.sync_copy(acc.at[pl.ds(r0, 624)],
                    out.at[pl.ds(r0, 624), pl.ds(c0, DH)])

    @pl.when(sid == NS - 1)
    def _():
        r1 = pl.multiple_of((NS - 1) * 624 + 624, 8)
        pltpu.sync_copy(acc.at[pl.ds(r1, 16)],
                        out.at[pl.ds(r1, 16), pl.ds(c0, DH)])


_mp_kernel = functools.partial(
    pl.kernel,
    mesh=plsc.VectorSubcoreMesh(core_axis_name="c", subcore_axis_name="s"),
    out_type=jax.ShapeDtypeStruct((N_NODES, D_FEAT), jnp.float32),
    scratch_types=[
        pltpu.VMEM((NBUF, BATCH), jnp.int32),
        pltpu.VMEM((NBUF, BATCH), jnp.int32),
        pltpu.VMEM((NBUF, BATCH, DH), jnp.float32),
        pltpu.VMEM((ZROWS, DH), jnp.float32),
        pltpu.VMEM_SHARED((N_NODES + PAD_ROWS, DH), jnp.float32),
    ] + [pltpu.SemaphoreType.DMA] * (4 * NBUF),
)(_mp_body)


@jax.jit
def kernel(x, edge_index):
    x2 = x.reshape(N_NODES * 2, DH)
    pad = jnp.arange(E_PAD - N_EDGES, dtype=jnp.int32)
    src4 = jnp.concatenate([edge_index[1], pad % N_NODES]).reshape(NS, NB, BATCH)
    dst4 = jnp.concatenate(
        [edge_index[0], N_NODES + pad % PAD_ROWS]).reshape(NS, NB, BATCH)
    return _mp_kernel(src4, dst4, x2)
